# baseline (speedup 1.0000x reference)
import functools

import jax
import jax.numpy as jnp
from jax import lax
from jax.experimental import pallas as pl
from jax.experimental.pallas import tpu as pltpu
from jax.experimental.pallas import tpu_sc as plsc

T_LEN = 50
B_LEN = 4096
D = 16
NSTATES = 1000000
NC, NS = 2, 16
BCOL = B_LEN // NS
JPC = D // NC
NVEC = (T_LEN * BCOL) // 16

_mesh = plsc.VectorSubcoreMesh(core_axis_name="c", subcore_axis_name="s")


def _gather_body(xt_hbm, mt_hbm, out_hbm, xv, xflat, outv, gsem, osem):
    c = lax.axis_index("c")
    s = lax.axis_index("s")
    col0 = s * BCOL

    pltpu.sync_copy(xt_hbm.at[:, pl.ds(col0, BCOL)], xv)

    def _flatten(i, carry):
        r = i // (BCOL // 16)
        k = i % (BCOL // 16)
        xflat[pl.ds(i * 16, 16)] = xv[r, pl.ds(k * 16, 16)]
        return carry

    lax.fori_loop(0, NVEC, _flatten, 0)

    out_copies = [None, None]
    for j in range(JPC):
        jj = c * JPC + j
        jb = j % 2
        gathers = []
        for r in range(T_LEN):
            cp = pltpu.async_copy(
                mt_hbm.at[jj].at[plsc.Indices(xflat.at[pl.ds(r * BCOL, BCOL)])],
                outv.at[jb, r],
                gsem,
            )
            gathers.append(cp)
        for cp in gathers:
            cp.wait()
        if out_copies[jb] is not None:
            out_copies[jb].wait()
        out_copies[jb] = pltpu.async_copy(
            outv.at[jb],
            out_hbm.at[:, jj, pl.ds(col0, BCOL)],
            osem,
        )
    for cp in out_copies:
        if cp is not None:
            cp.wait()


_sc_gather = functools.partial(
    pl.kernel,
    mesh=_mesh,
    out_type=jax.ShapeDtypeStruct((T_LEN, D, B_LEN), jnp.float32),
    scratch_types=[
        pltpu.VMEM((T_LEN, BCOL), jnp.int32),
        pltpu.VMEM((T_LEN * BCOL,), jnp.int32),
        pltpu.VMEM((2, T_LEN, BCOL), jnp.float32),
        pltpu.SemaphoreType.DMA,
        pltpu.SemaphoreType.DMA,
    ],
    compiler_params=pltpu.CompilerParams(use_tc_tiling_on_sc=False),
)(_gather_body)


def _scale_body(sig_ref, out_ref):
    sig = sig_ref[0, :]
    out_ref[...] = jnp.broadcast_to(sig[None, :, None], out_ref.shape)


def _scale_bcast(sigma):
    return pl.pallas_call(
        _scale_body,
        out_shape=jax.ShapeDtypeStruct((T_LEN, D, B_LEN), jnp.float32),
        grid=(T_LEN // 5,),
        in_specs=[pl.BlockSpec((1, D), lambda i: (0, 0))],
        out_specs=pl.BlockSpec((5, D, B_LEN), lambda i: (i, 0, 0)),
    )(sigma.reshape(1, D))


def kernel(x, u, t, means, sigma):
    xt = jnp.swapaxes(x.astype(jnp.int32), 0, 1)
    mt = jnp.swapaxes(means, 0, 1)
    outp = _sc_gather(xt, mt)
    loc = jnp.transpose(outp, (2, 0, 1))
    scale = jnp.transpose(_scale_bcast(sigma), (2, 0, 1))
    return (loc, scale)
